# Initial kernel scaffold; baseline (speedup 1.0000x reference)
import jax
import jax.numpy as jnp
from jax.experimental import pallas as pl


def kernel(x, va_rows, va_cols, va_vals):
    raise NotImplementedError("write your pallas kernel here")


# speedup vs baseline: 12.1629x; 12.1629x over previous
import functools

import jax
import jax.numpy as jnp
from jax import lax
from jax.experimental import pallas as pl
from jax.experimental.pallas import tpu as pltpu
from jax.experimental.pallas import tpu_sc as plsc

NNZ_PER_ROW = 7
ROWS_PER_CHUNK = 8
IDX_PER_CHUNK = NNZ_PER_ROW * ROWS_PER_CHUNK
LANES = 16


def _sc_spmm(xt, cols, vals, n_chunks, d):
    info = plsc.get_sparse_core_info()
    n_workers = info.num_cores * info.num_subcores
    num_cores = info.num_cores
    d_slices = d // LANES
    iters = (n_chunks + n_workers - 1) // n_workers
    mesh = plsc.VectorSubcoreMesh(core_axis_name="c", subcore_axis_name="s")

    @functools.partial(
        pl.kernel,
        mesh=mesh,
        out_type=jax.ShapeDtypeStruct((n_chunks * ROWS_PER_CHUNK, d),
                                      jnp.float32),
        scratch_types=[
            pltpu.VMEM((IDX_PER_CHUNK,), jnp.int32),
            pltpu.VMEM((IDX_PER_CHUNK + LANES,), jnp.float32),
            pltpu.VMEM((IDX_PER_CHUNK, d), jnp.float32),
            pltpu.VMEM((ROWS_PER_CHUNK, d), jnp.float32),
            pltpu.SemaphoreType.DMA,
        ],
    )
    def k(xt_hbm, cols_hbm, vals_hbm, out_hbm, idx_v, w_v, gath_v, outc_v,
          sem):
        wid = lax.axis_index("s") * num_cores + lax.axis_index("c")

        def chunk_body(i, _):
            c = i * n_workers + wid

            @pl.when(c < n_chunks)
            def _():
                nnz_base = c * IDX_PER_CHUNK
                pltpu.sync_copy(cols_hbm.at[pl.ds(nnz_base, IDX_PER_CHUNK)],
                                idx_v)
                pltpu.sync_copy(vals_hbm.at[pl.ds(nnz_base, IDX_PER_CHUNK)],
                                w_v.at[pl.ds(0, IDX_PER_CHUNK)])
                pltpu.async_copy(xt_hbm.at[idx_v], gath_v, sem).wait()

                def row_body(j, _):
                    base = j * NNZ_PER_ROW
                    w_vec = w_v[pl.ds(base, LANES)]

                    def col_body(v, _):
                        sl = pl.ds(v * LANES, LANES)
                        acc = w_vec[0] * gath_v[base, sl]
                        for kk in range(1, NNZ_PER_ROW):
                            acc = acc + w_vec[kk] * gath_v[base + kk, sl]
                        outc_v[j, sl] = acc
                        return 0

                    lax.fori_loop(0, d_slices, col_body, 0)
                    return 0

                lax.fori_loop(0, ROWS_PER_CHUNK, row_body, 0)
                pltpu.sync_copy(
                    outc_v,
                    out_hbm.at[pl.ds(c * ROWS_PER_CHUNK, ROWS_PER_CHUNK)])

            return 0

        lax.fori_loop(0, iters, chunk_body, 0)

    return k(xt, cols, vals)


def kernel(x, va_rows, va_cols, va_vals):
    b, ch, v_in = x.shape
    d = b * ch
    nnz = va_cols.shape[0]
    v_out = nnz // NNZ_PER_ROW
    n_chunks = (v_out + ROWS_PER_CHUNK - 1) // ROWS_PER_CHUNK
    pad = n_chunks * IDX_PER_CHUNK - nnz

    xt = jnp.transpose(x.reshape(d, v_in))
    cols_p = jnp.concatenate(
        [va_cols, jnp.zeros((pad,), jnp.int32)]) if pad else va_cols
    vals_p = jnp.concatenate(
        [va_vals, jnp.zeros((pad,), jnp.float32)]) if pad else va_vals

    out_t = _sc_spmm(xt, cols_p, vals_p, n_chunks, d)
    return jnp.transpose(out_t[:v_out]).reshape(b, ch, v_out)


# speedup vs baseline: 12.3307x; 1.0138x over previous
import functools

import jax
import jax.numpy as jnp
from jax import lax
from jax.experimental import pallas as pl
from jax.experimental.pallas import tpu as pltpu
from jax.experimental.pallas import tpu_sc as plsc

NNZ_PER_ROW = 7
ROWS_PER_CHUNK = 8
IDX_PER_CHUNK = NNZ_PER_ROW * ROWS_PER_CHUNK
LANES = 16


def _sc_spmm(xt, cols_t, vals_t, iters, n_workers, num_cores, d):
    d_slices = d // LANES
    idx_per_tile = iters * IDX_PER_CHUNK
    w_per_tile = idx_per_tile + LANES
    mesh = plsc.VectorSubcoreMesh(core_axis_name="c", subcore_axis_name="s")

    @functools.partial(
        pl.kernel,
        mesh=mesh,
        out_type=jax.ShapeDtypeStruct(
            (n_workers * iters * ROWS_PER_CHUNK, d), jnp.float32),
        scratch_types=[
            pltpu.VMEM((idx_per_tile,), jnp.int32),
            pltpu.VMEM((w_per_tile,), jnp.float32),
            pltpu.VMEM((2, IDX_PER_CHUNK, d), jnp.float32),
            pltpu.VMEM((ROWS_PER_CHUNK, d), jnp.float32),
            pltpu.SemaphoreType.DMA,
            pltpu.SemaphoreType.DMA,
        ],
    )
    def k(xt_hbm, cols_hbm, vals_hbm, out_hbm, idx_v, w_v, gath_v, outc_v,
          sem0, sem1):
        wid = lax.axis_index("s") * num_cores + lax.axis_index("c")
        sems = (sem0, sem1)

        pltpu.sync_copy(cols_hbm.at[pl.ds(wid * idx_per_tile, idx_per_tile)],
                        idx_v)
        pltpu.sync_copy(vals_hbm.at[pl.ds(wid * w_per_tile, w_per_tile)],
                        w_v)

        def gather_start(i, p):
            pltpu.async_copy(
                xt_hbm.at[idx_v.at[pl.ds(i * IDX_PER_CHUNK, IDX_PER_CHUNK)]],
                gath_v.at[p], sems[p])

        def gather_wait(i, p):
            pltpu.make_async_copy(
                xt_hbm.at[idx_v.at[pl.ds(i * IDX_PER_CHUNK, IDX_PER_CHUNK)]],
                gath_v.at[p], sems[p]).wait()

        def compute(i, p):
            gb = gath_v.at[p]

            def row_body(j, _):
                base = j * NNZ_PER_ROW
                w_vec = w_v[pl.ds(i * IDX_PER_CHUNK + base, LANES)]

                def col_body(v, _):
                    sl = pl.ds(v * LANES, LANES)
                    acc = w_vec[0] * gb[base, sl]
                    for kk in range(1, NNZ_PER_ROW):
                        acc = acc + w_vec[kk] * gb[base + kk, sl]
                    outc_v[j, sl] = acc
                    return 0

                lax.fori_loop(0, d_slices, col_body, 0, unroll=2)
                return 0

            lax.fori_loop(0, ROWS_PER_CHUNK, row_body, 0)
            c = i * n_workers + wid
            pltpu.sync_copy(
                outc_v,
                out_hbm.at[pl.ds(c * ROWS_PER_CHUNK, ROWS_PER_CHUNK)])

        gather_start(0, 0)
        gather_start(1, 1)

        def chunk_pair(i2, _):
            for p in range(2):
                i = i2 + p
                gather_wait(i, p)
                compute(i, p)

                @pl.when(i + 2 < iters)
                def _():
                    gather_start(i + 2, p)
            return 0

        assert iters % 2 == 0
        lax.fori_loop(0, iters // 2, lambda h, a: chunk_pair(h * 2, a), 0)

    return k(xt, cols_t, vals_t)


def kernel(x, va_rows, va_cols, va_vals):
    b, ch, v_in = x.shape
    d = b * ch
    nnz = va_cols.shape[0]
    v_out = nnz // NNZ_PER_ROW

    info = plsc.get_sparse_core_info()
    n_workers = info.num_cores * info.num_subcores
    n_chunks = (v_out + ROWS_PER_CHUNK - 1) // ROWS_PER_CHUNK
    iters = (n_chunks + n_workers - 1) // n_workers
    iters = iters + (iters % 2)
    n_chunks_pad = iters * n_workers
    pad = n_chunks_pad * IDX_PER_CHUNK - nnz

    xt = jnp.transpose(x.reshape(d, v_in))
    cols_p = jnp.concatenate([va_cols, jnp.zeros((pad,), jnp.int32)])
    vals_p = jnp.concatenate([va_vals, jnp.zeros((pad,), jnp.float32)])
    cols_t = jnp.transpose(
        cols_p.reshape(iters, n_workers, IDX_PER_CHUNK),
        (1, 0, 2)).reshape(n_workers * iters * IDX_PER_CHUNK)
    vals_t = jnp.pad(
        jnp.transpose(vals_p.reshape(iters, n_workers, IDX_PER_CHUNK),
                      (1, 0, 2)).reshape(n_workers, iters * IDX_PER_CHUNK),
        ((0, 0), (0, LANES))).reshape(-1)

    out_t = _sc_spmm(xt, cols_t, vals_t, iters, n_workers, info.num_cores, d)
    return jnp.transpose(out_t[:v_out]).reshape(b, ch, v_out)
